# baseline (speedup 1.0000x reference)
import functools

import jax
import jax.numpy as jnp
from jax import lax
from jax.experimental import pallas as pl
from jax.experimental.pallas import tpu as pltpu
from jax.experimental.pallas import tpu_sc as plsc

NV = 10000
D = 128
NNZ = 320000
TR = 10240
NT = 16
B = 128
GC = 160
NG = 16
NGROUPS = GC // NG
SPC = GC * B
P = 16 * SPC
GD = 160
SPD = GD * B
HT = TR // 2
HPT = HT // NT
AROWS = HT + 128
APT = AROWS // NT
ROWS_PT = TR // NT
WB = ROWS_PT // B
DUMP = 10000
BLK = 2048
BLK2 = 1024
NBUF = 2

_mesh = plsc.VectorSubcoreMesh(core_axis_name="c", subcore_axis_name="s")


def _zero_rows(buf, nrows, ncols):
    z = jnp.zeros((16,), jnp.float32)

    def body(r, _):
        for l in range(ncols // 16):
            buf[r, pl.ds(l * 16, 16)] = z
        return 0

    lax.fori_loop(0, nrows, body, 0)


@functools.partial(
    pl.kernel,
    out_type=jax.ShapeDtypeStruct((2 * TR,), jnp.float32),
    mesh=_mesh,
    scratch_types=[
        pltpu.VMEM((B,), jnp.int32),
        pltpu.VMEM((B,), jnp.float32),
        pltpu.VMEM_SHARED((TR,), jnp.float32),
        pltpu.SemaphoreType.DMA,
    ],
)
def _deg_kernel(didx, deg_out, idx_v, ones_v, acc, sem):
    c = lax.axis_index("c")
    t = lax.axis_index("s")
    base_r = t * ROWS_PT

    def fill(val):
        v = jnp.full((16,), val, jnp.float32)

        def body(i, _):
            ones_v[pl.ds(i * 16, 16)] = v
            return 0

        lax.fori_loop(0, B // 16, body, 0)

    fill(0.0)
    for k in range(WB):
        pltpu.sync_copy(ones_v, acc.at[pl.ds(base_r + k * B, B)])
    fill(1.0)
    plsc.subcore_barrier()

    ib = c * P + t * SPD

    def body(g, _):
        pltpu.sync_copy(didx.at[pl.ds(ib + g * B, B)], idx_v)
        pltpu.sync_copy(ones_v, acc.at[idx_v], add=True)
        return 0

    lax.fori_loop(0, GD, body, 0)
    plsc.subcore_barrier()
    pltpu.sync_copy(acc.at[pl.ds(base_r, ROWS_PT)],
                    deg_out.at[pl.ds(c * TR + base_r, ROWS_PT)])


@functools.partial(
    pl.kernel,
    out_type=jax.ShapeDtypeStruct((4 * HT, D), jnp.float32),
    mesh=_mesh,
    scratch_types=[
        pltpu.VMEM((2, NG, B), jnp.int32),
        pltpu.VMEM((2, NG, B), jnp.int32),
        pltpu.VMEM((NBUF, B, D), jnp.float32),
        pltpu.VMEM_SHARED((HT + 128, D), jnp.float32),
        pltpu.VMEM_SHARED((AROWS, D), jnp.float32),
        pltpu.SemaphoreType.DMA,
        pltpu.SemaphoreType.DMA,
        pltpu.SemaphoreType.DMA,
    ],
)
def _sweep_kernel(table, g2d, s2d, part, gbuf, sbuf, rows,
                  tab, acc, sem_g, sem_s, sem_i):
    c = lax.axis_index("c")
    t = lax.axis_index("s")
    crow = t * GC
    toff = c * HT

    sbase = t * HPT
    for (off, nr) in ((0, B), (B, B), (2 * B, HPT - 2 * B)):
        pltpu.sync_copy(table.at[pl.ds(toff + sbase + off, nr)],
                        rows.at[0, pl.ds(0, nr)])
        pltpu.sync_copy(rows.at[0, pl.ds(0, nr)],
                        tab.at[pl.ds(sbase + off, nr)])
    _zero_rows(rows.at[1], B, D)
    pltpu.sync_copy(rows.at[1, pl.ds(0, 8)], tab.at[pl.ds(HT + t * 8, 8)])

    def zero_acc():
        zbase = t * APT
        for (off, nr) in ((0, B), (B, B), (2 * B, APT - 2 * B)):
            eff = nr
            pltpu.sync_copy(rows.at[1, pl.ds(0, eff)],
                            acc.at[pl.ds(zbase + off, eff)])

    zero_acc()
    plsc.subcore_barrier()

    for p in range(2):
        d = lax.rem(c + p, 2)
        IR = P // B + NG
        crow_g = c * IR + t * GC
        crow_s = d * IR + t * GC

        pltpu.sync_copy(g2d.at[pl.ds(crow_g, NG)], gbuf.at[0])
        pltpu.sync_copy(s2d.at[pl.ds(crow_s, NG)], sbuf.at[0])

        def group(m, _):
            pm = lax.rem(m, 2)
            pn = lax.rem(m + 1, 2)
            ig = pltpu.async_copy(g2d.at[pl.ds(crow_g + (m + 1) * NG, NG)],
                                  gbuf.at[pn], sem_i)
            is_ = pltpu.async_copy(s2d.at[pl.ds(crow_s + (m + 1) * NG, NG)],
                                   sbuf.at[pn], sem_i)
            gb = gbuf.at[pm]
            sb = sbuf.at[pm]
            dg = [None] * NG
            sc = [None] * NG
            for j in range(NBUF):
                dg[j] = pltpu.async_copy(tab.at[gb.at[j]], rows.at[j], sem_g)
            for j in range(NG):
                dg[j].wait()
                sc[j] = pltpu.async_copy(rows.at[j % NBUF],
                                         acc.at[sb.at[j]], sem_s, add=True)
                if j + NBUF < NG:
                    sc[j].wait()
                    dg[j + NBUF] = pltpu.async_copy(
                        tab.at[gb.at[j + NBUF]], rows.at[j % NBUF], sem_g)
            for j in range(NG - NBUF, NG):
                sc[j].wait()
            ig.wait()
            is_.wait()
            return 0

        lax.fori_loop(0, NGROUPS, group, 0)
        plsc.subcore_barrier()

        obase = (c * 2 + p) * HT + t * HPT
        for (off, nr) in ((0, B), (B, B), (2 * B, HPT - 2 * B)):
            pltpu.sync_copy(acc.at[pl.ds(t * HPT + off, nr)],
                            part.at[pl.ds(obase + off, nr)])
        if p == 0:
            plsc.subcore_barrier()
            _zero_rows(rows.at[1], B, D)
            zero_acc()
            plsc.subcore_barrier()


def _tc_scales(deg2):
    def body(dref, oref):
        d = dref[...]
        safe = jnp.where(d > 0, d, 1.0)
        row = lax.broadcasted_iota(jnp.int32, (2 * TR // 128, 128), 0)
        oref[...] = jnp.where(row < TR // 128, lax.rsqrt(safe), 1.0 / safe)

    return pl.pallas_call(
        body,
        out_shape=jax.ShapeDtypeStruct((2 * TR // 128, 128), jnp.float32),
    )(deg2)


def _tc_xs(x, dvs_col):
    def body(xref, dref, oref):
        oref[...] = xref[...] * dref[...]

    return pl.pallas_call(
        body,
        grid=(TR // BLK,),
        in_specs=[pl.BlockSpec((BLK, D), lambda g: (g, 0)),
                  pl.BlockSpec((BLK, 1), lambda g: (g, 0))],
        out_specs=pl.BlockSpec((BLK, D), lambda g: (g, 0)),
        out_shape=jax.ShapeDtypeStruct((TR, D), jnp.float32),
    )(x, dvs_col)


def _tc_hemerge(part, de_col):
    def body(aref, bref, dref, oref):
        oref[...] = (aref[...] + bref[...]) * dref[...]

    nb = TR // BLK2 // 2
    return pl.pallas_call(
        body,
        grid=(2 * nb,),
        in_specs=[pl.BlockSpec((BLK2, D),
                               lambda g: (jnp.where(g < nb, g, g + nb), 0)),
                  pl.BlockSpec((BLK2, D),
                               lambda g: (jnp.where(g < nb, g + 3 * nb, g), 0)),
                  pl.BlockSpec((BLK2, 1), lambda g: (g, 0))],
        out_specs=pl.BlockSpec((BLK2, D), lambda g: (g, 0)),
        out_shape=jax.ShapeDtypeStruct((TR, D), jnp.float32),
    )(part, part, de_col)


def _tc_layer(xp, agg_part, dvs_col, W, b2d):
    def body(xref, aref, bref, dref, wref, biasref, o1, o2):
        a = (aref[...] + bref[...]) * dref[...]
        y = jnp.dot(a, wref[...], preferred_element_type=jnp.float32)
        xn = jnp.maximum(xref[...] + y + biasref[...], 0.0)
        o1[...] = xn
        o2[...] = xn * dref[...]

    nb = TR // BLK2 // 2
    return pl.pallas_call(
        body,
        grid=(2 * nb,),
        in_specs=[pl.BlockSpec((BLK2, D), lambda g: (g, 0)),
                  pl.BlockSpec((BLK2, D),
                               lambda g: (jnp.where(g < nb, g, g + nb), 0)),
                  pl.BlockSpec((BLK2, D),
                               lambda g: (jnp.where(g < nb, g + 3 * nb, g), 0)),
                  pl.BlockSpec((BLK2, 1), lambda g: (g, 0)),
                  pl.BlockSpec((D, D), lambda g: (0, 0)),
                  pl.BlockSpec((1, D), lambda g: (0, 0))],
        out_specs=[pl.BlockSpec((BLK2, D), lambda g: (g, 0))] * 2,
        out_shape=(jax.ShapeDtypeStruct((TR, D), jnp.float32),) * 2,
    )(xp, agg_part, agg_part, dvs_col, W, b2d)


def kernel(node_features, incidence, W1, b1, W2, b2):
    nidx = incidence[0]
    eidx = incidence[1]
    pad = jnp.full((P - NNZ,), DUMP, jnp.int32)
    padrows = jnp.full((NG, B), 0, jnp.int32)
    nidx_p = jnp.concatenate([nidx, pad])
    eidx_p = jnp.concatenate([eidx, pad])
    didx = jnp.concatenate([nidx_p, eidx_p])
    spread = HT + (jnp.arange(P, dtype=jnp.int32) % 128)

    def halfpack(idx_p):
        packs = []
        for h in (0, 1):
            m = idx_p - h * HT
            v = jnp.where((m >= 0) & (m < HT), m, spread)
            packs.append(jnp.concatenate([v.reshape(P // B, B), padrows]))
        return jnp.concatenate(packs)

    nidx2 = halfpack(nidx_p)
    eidx2 = halfpack(eidx_p)
    x_pad = jnp.concatenate(
        [node_features, jnp.zeros((TR - NV, D), jnp.float32)], axis=0)

    deg = _deg_kernel(didx)
    scales = _tc_scales(deg.reshape(2 * TR // 128, 128))
    sflat = scales.reshape(-1)
    dvs_col = sflat[:TR, None]
    de_col = sflat[TR:, None]

    xs = _tc_xs(x_pad, dvs_col)
    xp = x_pad
    for (W, b) in ((W1, b1), (W2, b2)):
        he_part = _sweep_kernel(xs, nidx2, eidx2)
        he = _tc_hemerge(he_part, de_col)
        agg_part = _sweep_kernel(he, eidx2, nidx2)
        xp, xs = _tc_layer(xp, agg_part, dvs_col, W, b.reshape(1, D))
    return xp[:NV]


# speedup vs baseline: 1.3735x; 1.3735x over previous
import functools

import jax
import jax.numpy as jnp
from jax import lax
from jax.experimental import pallas as pl
from jax.experimental.pallas import tpu as pltpu
from jax.experimental.pallas import tpu_sc as plsc

NV = 10000
D = 128
NNZ = 320000
TR = 10240
NT = 16
B = 128
GC = 79
SPC = GC * B
P = 32 * SPC
GD = 158
SPD = GD * B
ROWS_PT = TR // NT
WB = ROWS_PT // B
DUMP = 10000
BLK = 2048

_mesh = plsc.VectorSubcoreMesh(core_axis_name="c", subcore_axis_name="s")


def _zero_rows(buf, nrows, ncols):
    z = jnp.zeros((16,), jnp.float32)

    def body(r, _):
        for l in range(ncols // 16):
            buf[r, pl.ds(l * 16, 16)] = z
        return 0

    lax.fori_loop(0, nrows, body, 0)


@functools.partial(
    pl.kernel,
    out_type=jax.ShapeDtypeStruct((2 * TR,), jnp.float32),
    mesh=_mesh,
    scratch_types=[
        pltpu.VMEM((B,), jnp.int32),
        pltpu.VMEM((B,), jnp.float32),
        pltpu.VMEM_SHARED((TR,), jnp.float32),
        pltpu.SemaphoreType.DMA,
    ],
)
def _deg_kernel(didx, deg_out, idx_v, ones_v, acc, sem):
    c = lax.axis_index("c")
    t = lax.axis_index("s")
    base_r = t * ROWS_PT

    def fill(val):
        v = jnp.full((16,), val, jnp.float32)

        def body(i, _):
            ones_v[pl.ds(i * 16, 16)] = v
            return 0

        lax.fori_loop(0, B // 16, body, 0)

    fill(0.0)
    for k in range(WB):
        pltpu.sync_copy(ones_v, acc.at[pl.ds(base_r + k * B, B)])
    fill(1.0)
    plsc.subcore_barrier()

    ib = c * P + t * SPD

    def body(g, _):
        pltpu.sync_copy(didx.at[pl.ds(ib + g * B, B)], idx_v)
        pltpu.sync_copy(ones_v, acc.at[idx_v], add=True)
        return 0

    lax.fori_loop(0, GD, body, 0)
    plsc.subcore_barrier()
    pltpu.sync_copy(acc.at[pl.ds(base_r, ROWS_PT)],
                    deg_out.at[pl.ds(c * TR + base_r, ROWS_PT)])


@functools.partial(
    pl.kernel,
    out_type=jax.ShapeDtypeStruct((2 * TR, D), jnp.float32),
    mesh=_mesh,
    scratch_types=[
        pltpu.VMEM((B,), jnp.int32),
        pltpu.VMEM((B,), jnp.int32),
        pltpu.VMEM((B, D), jnp.float32),
        pltpu.VMEM((B, D), jnp.float32),
        pltpu.VMEM_SHARED((TR, D), jnp.float32),
        pltpu.SemaphoreType.DMA,
    ],
)
def _sweep_kernel(table, gidx, sidx, part, gidx_v, sidx_v, rows_v, zb_v,
                  acc, sem):
    c = lax.axis_index("c")
    t = lax.axis_index("s")
    base_r = t * ROWS_PT

    _zero_rows(zb_v, B, D)
    for k in range(WB):
        pltpu.sync_copy(zb_v, acc.at[pl.ds(base_r + k * B, B)])
    plsc.subcore_barrier()

    ib = (c * NT + t) * SPC

    def body(g, _):
        pltpu.sync_copy(gidx.at[pl.ds(ib + g * B, B)], gidx_v)
        pltpu.sync_copy(sidx.at[pl.ds(ib + g * B, B)], sidx_v)
        pltpu.async_copy(table.at[gidx_v], rows_v, sem).wait()
        pltpu.sync_copy(rows_v, acc.at[sidx_v], add=True)
        return 0

    lax.fori_loop(0, GC, body, 0)
    plsc.subcore_barrier()
    pltpu.sync_copy(acc.at[pl.ds(base_r, ROWS_PT)],
                    part.at[pl.ds(c * TR + base_r, ROWS_PT)])


def _tc_scales(deg2):
    def body(dref, oref):
        d = dref[...]
        safe = jnp.where(d > 0, d, 1.0)
        row = lax.broadcasted_iota(jnp.int32, (2 * TR // 128, 128), 0)
        oref[...] = jnp.where(row < TR // 128, lax.rsqrt(safe), 1.0 / safe)

    return pl.pallas_call(
        body,
        out_shape=jax.ShapeDtypeStruct((2 * TR // 128, 128), jnp.float32),
    )(deg2)


def _tc_xs(x, dvs_col):
    def body(xref, dref, oref):
        oref[...] = xref[...] * dref[...]

    return pl.pallas_call(
        body,
        grid=(TR // BLK,),
        in_specs=[pl.BlockSpec((BLK, D), lambda g: (g, 0)),
                  pl.BlockSpec((BLK, 1), lambda g: (g, 0))],
        out_specs=pl.BlockSpec((BLK, D), lambda g: (g, 0)),
        out_shape=jax.ShapeDtypeStruct((TR, D), jnp.float32),
    )(x, dvs_col)


def _tc_hemerge(part, de_col):
    def body(aref, bref, dref, oref):
        oref[...] = (aref[...] + bref[...]) * dref[...]

    return pl.pallas_call(
        body,
        grid=(TR // BLK,),
        in_specs=[pl.BlockSpec((BLK, D), lambda g: (g, 0)),
                  pl.BlockSpec((BLK, D), lambda g: (g + TR // BLK, 0)),
                  pl.BlockSpec((BLK, 1), lambda g: (g, 0))],
        out_specs=pl.BlockSpec((BLK, D), lambda g: (g, 0)),
        out_shape=jax.ShapeDtypeStruct((TR, D), jnp.float32),
    )(part, part, de_col)


def _tc_layer(xp, agg_part, dvs_col, W, b2d):
    def body(xref, aref, bref, dref, wref, biasref, o1, o2):
        a = (aref[...] + bref[...]) * dref[...]
        y = jnp.dot(a, wref[...], preferred_element_type=jnp.float32)
        xn = jnp.maximum(xref[...] + y + biasref[...], 0.0)
        o1[...] = xn
        o2[...] = xn * dref[...]

    return pl.pallas_call(
        body,
        grid=(TR // BLK,),
        in_specs=[pl.BlockSpec((BLK, D), lambda g: (g, 0)),
                  pl.BlockSpec((BLK, D), lambda g: (g, 0)),
                  pl.BlockSpec((BLK, D), lambda g: (g + TR // BLK, 0)),
                  pl.BlockSpec((BLK, 1), lambda g: (g, 0)),
                  pl.BlockSpec((D, D), lambda g: (0, 0)),
                  pl.BlockSpec((1, D), lambda g: (0, 0))],
        out_specs=[pl.BlockSpec((BLK, D), lambda g: (g, 0))] * 2,
        out_shape=(jax.ShapeDtypeStruct((TR, D), jnp.float32),) * 2,
    )(xp, agg_part, agg_part, dvs_col, W, b2d)


def kernel(node_features, incidence, W1, b1, W2, b2):
    nidx = incidence[0]
    eidx = incidence[1]
    pad = jnp.full((P - NNZ,), DUMP, jnp.int32)
    nidx_p = jnp.concatenate([nidx, pad])
    eidx_p = jnp.concatenate([eidx, pad])
    didx = jnp.concatenate([nidx_p, eidx_p])
    x_pad = jnp.concatenate(
        [node_features, jnp.zeros((TR - NV, D), jnp.float32)], axis=0)

    deg = _deg_kernel(didx)
    scales = _tc_scales(deg.reshape(2 * TR // 128, 128))
    sflat = scales.reshape(-1)
    dvs_col = sflat[:TR, None]
    de_col = sflat[TR:, None]

    xs = _tc_xs(x_pad, dvs_col)
    xp = x_pad
    for (W, b) in ((W1, b1), (W2, b2)):
        he_part = _sweep_kernel(xs, nidx_p, eidx_p)
        he = _tc_hemerge(he_part, de_col)
        agg_part = _sweep_kernel(he, eidx_p, nidx_p)
        xp, xs = _tc_layer(xp, agg_part, dvs_col, W, b.reshape(1, D))
    return xp[:NV]
